# Initial kernel scaffold; baseline (speedup 1.0000x reference)
import jax
import jax.numpy as jnp
from jax.experimental import pallas as pl


def kernel(x, conv_w, conv_b, gamma, beta):
    raise NotImplementedError("write your pallas kernel here")


# speedup vs baseline: 76.4913x; 76.4913x over previous
import jax
import jax.numpy as jnp
from jax import lax
from jax.experimental import pallas as pl
from jax.experimental.pallas import tpu as pltpu

B = 8
CIN = 3
H = W = 224
CO = 128
KHW = 7
PAD = 3
HT = 8
NH = H // HT
HP = H + 2 * PAD
KC = CIN * KHW * KHW
SLAB = (HT + KHW - 1) * CIN * KHW
NPIX = H * W
TOPK = 128
NEG = -1e30


def _conv_stats_kernel(xs_ref, wbig_ref, y_ref, sums_ref):
    h = pl.program_id(1)
    slab = xs_ref[0, pl.ds(h * HT * 21, SLAB)]
    rows = []
    for r in range(HT):
        a = slab[r * 21:r * 21 + KC]
        rows.append(lax.dot_general(
            wbig_ref[...], a, (((1,), (0,)), ((), ())),
            preferred_element_type=jnp.float32))
    y3 = jnp.stack(rows, axis=0)
    y_ref[0] = y3
    s1 = jnp.sum(y3, axis=0)
    s2 = jnp.sum(y3 * y3, axis=0)

    @pl.when(h == 0)
    def _init():
        sums_ref[0, 0] = s1
        sums_ref[0, 1] = s2

    @pl.when(h > 0)
    def _acc():
        sums_ref[0, 0] += s1
        sums_ref[0, 1] += s2


def _norm_max_kernel(y_ref, scale_ref, off_ref, mx_ref):
    y = y_ref[0]
    sc = scale_ref[...].reshape(1, CO, 1)
    of = off_ref[...].reshape(1, CO, 1)
    n = jnp.maximum(y * sc + of, 0.0)
    mx_ref[0] = jnp.max(n, axis=1)


def _topk_kernel(mx_ref, idx_ref, buf_ref):
    buf_ref[...] = mx_ref[0]
    rows = NPIX // 128
    pix = (lax.broadcasted_iota(jnp.int32, (rows, 128), 0) * 128
           + lax.broadcasted_iota(jnp.int32, (rows, 128), 1))
    lane = lax.broadcasted_iota(jnp.int32, (1, TOPK), 1)

    def body(r, acc):
        m = buf_ref[...]
        mv = jnp.max(m)
        p = jnp.min(jnp.where(m >= mv, pix, jnp.int32(2**30)))
        buf_ref[...] = jnp.where(pix == p, NEG, m)
        return jnp.where(lane == r, p, acc)

    acc = lax.fori_loop(0, TOPK, body, jnp.zeros((1, TOPK), jnp.int32))
    idx_ref[0, 0] = acc[0]


def _gather_top3_kernel(idx_sref, y_hbm, scale_ref, off_ref,
                        vals_ref, chf_ref, rows_ref, sem):
    b = pl.program_id(0)
    lane = lax.broadcasted_iota(jnp.int32, (CO, TOPK), 1)
    lane_w = lax.broadcasted_iota(jnp.int32, (CO, W), 1)

    def row_copy(p, slot):
        hh = idx_sref[b, p] // W
        return pltpu.make_async_copy(
            y_hbm.at[b, hh], rows_ref.at[slot], sem)

    row_copy(0, 0).start()

    def body(p, g):
        @pl.when(p < TOPK - 1)
        def _():
            row_copy(p + 1, (p + 1) % 2).start()
        row_copy(p, p % 2).wait()
        t = idx_sref[b, p]
        ww = t - (t // W) * W
        row = rows_ref[pl.ds(p % 2, 1)].reshape(CO, W)
        col = jnp.sum(jnp.where(lane_w == ww, row, 0.0), axis=1, keepdims=True)
        return g + jnp.where(lane == p, col, 0.0)

    g = lax.fori_loop(0, TOPK, body, jnp.zeros((CO, TOPK), jnp.float32))
    g = jnp.maximum(g * scale_ref[...] + off_ref[...], 0.0)
    ch_iota = lax.broadcasted_iota(jnp.int32, (CO, TOPK), 0)
    for r in range(3):
        mv = jnp.max(g, axis=0, keepdims=True)
        ch = jnp.min(jnp.where(g >= mv, ch_iota, jnp.int32(9999)),
                     axis=0, keepdims=True)
        vals_ref[0, r] = mv[0]
        chf_ref[0, r] = ch[0].astype(jnp.float32)
        g = jnp.where(ch_iota == ch, NEG, g)


def _forward(x, conv_w, conv_b, gamma, beta, debug=False):
    f32 = jnp.float32
    xp = jnp.pad(x, ((0, 0), (0, 0), (PAD, PAD), (PAD, PAD)))
    xpt = jnp.transpose(xp, (0, 2, 1, 3))
    xs = jnp.stack([xpt[:, :, :, dw:dw + W] for dw in range(KHW)],
                   axis=2).astype(jnp.bfloat16).reshape(B, HP * 21, W)
    wmat = jnp.transpose(conv_w, (0, 2, 3, 1)).reshape(CO, KC)
    wbig = wmat.astype(jnp.bfloat16)

    y, sums = pl.pallas_call(
        _conv_stats_kernel,
        grid=(B, NH),
        in_specs=[
            pl.BlockSpec((1, HP * 21, W), lambda b, h: (b, 0, 0)),
            pl.BlockSpec((CO, KC), lambda b, h: (0, 0)),
        ],
        out_specs=[
            pl.BlockSpec((1, HT, CO, W), lambda b, h: (b, h, 0, 0)),
            pl.BlockSpec((1, 2, CO, W), lambda b, h: (b, 0, 0, 0)),
        ],
        out_shape=[
            jax.ShapeDtypeStruct((B, H, CO, W), f32),
            jax.ShapeDtypeStruct((B, 2, CO, W), f32),
        ],
    )(xs, wbig)

    tot = jnp.sum(sums, axis=(0, 3))
    cnt = f32(B * H * W)
    rawmean = tot[0] / cnt
    var = tot[1] / cnt - rawmean ** 2
    scale = gamma * lax.rsqrt(var + 1e-5)
    off = beta - rawmean * scale
    scale_c = scale.reshape(CO, 1)
    off_c = off.reshape(CO, 1)

    mx = pl.pallas_call(
        _norm_max_kernel,
        grid=(B, NH),
        in_specs=[
            pl.BlockSpec((1, HT, CO, W), lambda b, h: (b, h, 0, 0)),
            pl.BlockSpec((CO, 1), lambda b, h: (0, 0)),
            pl.BlockSpec((CO, 1), lambda b, h: (0, 0)),
        ],
        out_specs=pl.BlockSpec((1, HT, W), lambda b, h: (b, h, 0)),
        out_shape=jax.ShapeDtypeStruct((B, H, W), f32),
    )(y, scale_c, off_c)
    mx2 = mx.reshape(B, NPIX // 128, 128)

    idx = pl.pallas_call(
        _topk_kernel,
        grid=(B,),
        in_specs=[pl.BlockSpec((1, NPIX // 128, 128), lambda b: (b, 0, 0))],
        out_specs=pl.BlockSpec((1, 1, TOPK), lambda b: (b, 0, 0)),
        out_shape=jax.ShapeDtypeStruct((B, 1, TOPK), jnp.int32),
        scratch_shapes=[pltpu.VMEM((NPIX // 128, 128), f32)],
    )(mx2)

    idx2 = idx.reshape(B, TOPK)
    vals, chf = pl.pallas_call(
        _gather_top3_kernel,
        grid=(B,),
        in_specs=[
            pl.BlockSpec(memory_space=pltpu.SMEM),
            pl.BlockSpec(memory_space=pl.ANY),
            pl.BlockSpec((CO, 1), lambda b: (0, 0)),
            pl.BlockSpec((CO, 1), lambda b: (0, 0)),
        ],
        out_specs=[
            pl.BlockSpec((1, 3, TOPK), lambda b: (b, 0, 0)),
            pl.BlockSpec((1, 3, TOPK), lambda b: (b, 0, 0)),
        ],
        out_shape=[
            jax.ShapeDtypeStruct((B, 3, TOPK), f32),
            jax.ShapeDtypeStruct((B, 3, TOPK), f32),
        ],
        scratch_shapes=[pltpu.VMEM((2, CO, W), f32), pltpu.SemaphoreType.DMA],
    )(idx2, y, scale_c, off_c)

    if debug:
        return (chf, vals, idx), dict(y=y, sums=sums, mx=mx, scale=scale,
                                      off=off, idx2=idx2)
    return (chf, vals, idx)


def kernel(x, conv_w, conv_b, gamma, beta):
    return _forward(x, conv_w, conv_b, gamma, beta)


# speedup vs baseline: 76.5013x; 1.0001x over previous
import jax
import jax.numpy as jnp
from jax import lax
from jax.experimental import pallas as pl
from jax.experimental.pallas import tpu as pltpu

B = 8
CIN = 3
H = W = 224
CO = 128
KHW = 7
PAD = 3
HT = 8
NH = H // HT
HP = H + 2 * PAD
KC = CIN * KHW * KHW
SLAB = (HT + KHW - 1) * CIN * KHW
NPIX = H * W
TOPK = 128
NEG = -1e30


def _conv_stats_kernel(xs_ref, wbig_ref, y_ref, sums_ref):
    h = pl.program_id(1)
    slab = xs_ref[0, pl.ds(h * HT * 21, SLAB)]
    rows = []
    for r in range(HT):
        a = slab[r * 21:r * 21 + KC]
        rows.append(lax.dot_general(
            wbig_ref[...], a, (((1,), (0,)), ((), ())),
            preferred_element_type=jnp.float32))
    y3 = jnp.stack(rows, axis=0)
    y_ref[0] = y3
    s1 = jnp.sum(y3, axis=0)
    s2 = jnp.sum(y3 * y3, axis=0)

    @pl.when(h == 0)
    def _init():
        sums_ref[0, 0] = s1
        sums_ref[0, 1] = s2

    @pl.when(h > 0)
    def _acc():
        sums_ref[0, 0] += s1
        sums_ref[0, 1] += s2


def _norm_max_kernel(y_ref, scale_ref, off_ref, mx_ref):
    y = y_ref[0]
    sc = scale_ref[...].reshape(1, CO, 1)
    of = off_ref[...].reshape(1, CO, 1)
    n = jnp.maximum(y * sc + of, 0.0)
    mx_ref[0] = jnp.max(n, axis=1)


def _topk_kernel(mx_ref, idx_ref, buf_ref):
    buf_ref[...] = mx_ref[0]
    rows = NPIX // 128
    pix = (lax.broadcasted_iota(jnp.int32, (rows, 128), 0) * 128
           + lax.broadcasted_iota(jnp.int32, (rows, 128), 1))
    lane = lax.broadcasted_iota(jnp.int32, (1, TOPK), 1)

    def body(r, acc):
        m = buf_ref[...]
        mv = jnp.max(m)
        p = jnp.min(jnp.where(m >= mv, pix, jnp.int32(2**30)))
        buf_ref[...] = jnp.where(pix == p, NEG, m)
        return jnp.where(lane == r, p, acc)

    acc = lax.fori_loop(0, TOPK, body, jnp.zeros((1, TOPK), jnp.int32))
    idx_ref[0, 0] = acc[0]


def _gather_top3_kernel(idx_sref, y_hbm, scale_ref, off_ref,
                        vals_ref, chf_ref, rows_ref, sem):
    b = pl.program_id(0)
    lane = lax.broadcasted_iota(jnp.int32, (CO, TOPK), 1)
    lane_w = lax.broadcasted_iota(jnp.int32, (CO, W), 1)

    def row_copy(p, slot):
        hh = idx_sref[b, p] // W
        return pltpu.make_async_copy(
            y_hbm.at[b, hh], rows_ref.at[slot], sem)

    row_copy(0, 0).start()

    def body(p, g):
        @pl.when(p < TOPK - 1)
        def _():
            row_copy(p + 1, (p + 1) % 2).start()
        row_copy(p, p % 2).wait()
        t = idx_sref[b, p]
        ww = t - (t // W) * W
        row = rows_ref[pl.ds(p % 2, 1)].reshape(CO, W)
        col = jnp.sum(jnp.where(lane_w == ww, row, 0.0), axis=1, keepdims=True)
        return g + jnp.where(lane == p, col, 0.0)

    g = lax.fori_loop(0, TOPK, body, jnp.zeros((CO, TOPK), jnp.float32))
    g = jnp.maximum(g * scale_ref[...] + off_ref[...], 0.0)
    ch_iota = lax.broadcasted_iota(jnp.int32, (CO, TOPK), 0)
    for r in range(3):
        mv = jnp.max(g, axis=0, keepdims=True)
        ch = jnp.min(jnp.where(g >= mv, ch_iota, jnp.int32(9999)),
                     axis=0, keepdims=True)
        vals_ref[0, r] = mv[0]
        chf_ref[0, r] = ch[0].astype(jnp.float32)
        g = jnp.where(ch_iota == ch, NEG, g)


def kernel(x, conv_w, conv_b, gamma, beta):
    f32 = jnp.float32
    xp = jnp.pad(x, ((0, 0), (0, 0), (PAD, PAD), (PAD, PAD)))
    xpt = jnp.transpose(xp, (0, 2, 1, 3))
    xs = jnp.stack([xpt[:, :, :, dw:dw + W] for dw in range(KHW)],
                   axis=2).astype(jnp.bfloat16).reshape(B, HP * 21, W)
    wmat = jnp.transpose(conv_w, (0, 2, 3, 1)).reshape(CO, KC)
    wbig = wmat.astype(jnp.bfloat16)

    y, sums = pl.pallas_call(
        _conv_stats_kernel,
        grid=(B, NH),
        in_specs=[
            pl.BlockSpec((1, HP * 21, W), lambda b, h: (b, 0, 0)),
            pl.BlockSpec((CO, KC), lambda b, h: (0, 0)),
        ],
        out_specs=[
            pl.BlockSpec((1, HT, CO, W), lambda b, h: (b, h, 0, 0)),
            pl.BlockSpec((1, 2, CO, W), lambda b, h: (b, 0, 0, 0)),
        ],
        out_shape=[
            jax.ShapeDtypeStruct((B, H, CO, W), f32),
            jax.ShapeDtypeStruct((B, 2, CO, W), f32),
        ],
    )(xs, wbig)

    tot = jnp.sum(sums, axis=(0, 3))
    cnt = f32(B * H * W)
    rawmean = tot[0] / cnt
    var = tot[1] / cnt - rawmean ** 2
    scale = gamma * lax.rsqrt(var + 1e-5)
    off = beta - rawmean * scale
    scale_c = scale.reshape(CO, 1)
    off_c = off.reshape(CO, 1)

    mx = pl.pallas_call(
        _norm_max_kernel,
        grid=(B, NH),
        in_specs=[
            pl.BlockSpec((1, HT, CO, W), lambda b, h: (b, h, 0, 0)),
            pl.BlockSpec((CO, 1), lambda b, h: (0, 0)),
            pl.BlockSpec((CO, 1), lambda b, h: (0, 0)),
        ],
        out_specs=pl.BlockSpec((1, HT, W), lambda b, h: (b, h, 0)),
        out_shape=jax.ShapeDtypeStruct((B, H, W), f32),
    )(y, scale_c, off_c)
    mx2 = mx.reshape(B, NPIX // 128, 128)

    idx = pl.pallas_call(
        _topk_kernel,
        grid=(B,),
        in_specs=[pl.BlockSpec((1, NPIX // 128, 128), lambda b: (b, 0, 0))],
        out_specs=pl.BlockSpec((1, 1, TOPK), lambda b: (b, 0, 0)),
        out_shape=jax.ShapeDtypeStruct((B, 1, TOPK), jnp.int32),
        scratch_shapes=[pltpu.VMEM((NPIX // 128, 128), f32)],
    )(mx2)

    idx2 = idx.reshape(B, TOPK)
    vals, chf = pl.pallas_call(
        _gather_top3_kernel,
        grid=(B,),
        in_specs=[
            pl.BlockSpec(memory_space=pltpu.SMEM),
            pl.BlockSpec(memory_space=pl.ANY),
            pl.BlockSpec((CO, 1), lambda b: (0, 0)),
            pl.BlockSpec((CO, 1), lambda b: (0, 0)),
        ],
        out_specs=[
            pl.BlockSpec((1, 3, TOPK), lambda b: (b, 0, 0)),
            pl.BlockSpec((1, 3, TOPK), lambda b: (b, 0, 0)),
        ],
        out_shape=[
            jax.ShapeDtypeStruct((B, 3, TOPK), f32),
            jax.ShapeDtypeStruct((B, 3, TOPK), f32),
        ],
        scratch_shapes=[pltpu.VMEM((2, CO, W), f32), pltpu.SemaphoreType.DMA],
    )(idx2, y, scale_c, off_c)

    return (chf, vals, idx)
